# Initial kernel scaffold; baseline (speedup 1.0000x reference)
import functools

import jax
import jax.numpy as jnp
from jax import lax
from jax.experimental import pallas as pl
from jax.experimental.pallas import tpu as pltpu
from jax.experimental.pallas import tpu_sc as plsc

DIM = 64
NC = 2
NS = 16
NW = NC * NS
CHUNK = 512


def _emb_body(table_hbm, idx_hbm, out_hbm, idx_v, rows_v, sem):
    wid = lax.axis_index("s") * NC + lax.axis_index("c")
    b_per_w = idx_hbm.shape[0] // NW
    n_chunks = b_per_w // CHUNK
    wbase = wid * b_per_w

    def body(i, carry):
        base = wbase + i * CHUNK
        pltpu.sync_copy(idx_hbm.at[pl.ds(base, CHUNK)], idx_v)
        pltpu.async_copy(table_hbm.at[idx_v], rows_v, sem).wait()
        pltpu.sync_copy(rows_v, out_hbm.at[pl.ds(base, CHUNK)])
        return carry

    lax.fori_loop(0, n_chunks, body, 0)


@jax.jit
def kernel(X, table):
    S, T = X.shape
    B = S * T
    idx = X.reshape(B).astype(jnp.int32)
    mesh = plsc.VectorSubcoreMesh(core_axis_name="c", subcore_axis_name="s")
    k = functools.partial(
        pl.kernel,
        mesh=mesh,
        out_type=jax.ShapeDtypeStruct((B, DIM), jnp.float32),
        scratch_types=[
            pltpu.VMEM((CHUNK,), jnp.int32),
            pltpu.VMEM((CHUNK, DIM), jnp.float32),
            pltpu.SemaphoreType.DMA,
        ],
    )(_emb_body)
    out = k(table, idx)
    return out.reshape(S, T, DIM)


# speedup vs baseline: 4.7455x; 4.7455x over previous
import functools

import jax
import jax.numpy as jnp
from jax import lax
from jax.experimental import pallas as pl
from jax.experimental.pallas import tpu as pltpu
from jax.experimental.pallas import tpu_sc as plsc

DIM = 64
NC = 2
NS = 16
NW = NC * NS
CHUNK = 512


def _emb_body(table_hbm, idx_hbm, out_hbm, idx_v, rows_v, sem):
    wid = lax.axis_index("s") * NC + lax.axis_index("c")
    b_per_w = idx_hbm.shape[0] // NW
    n_chunks = b_per_w // CHUNK
    wbase = wid * b_per_w

    def body(i, carry):
        base = wbase + i * CHUNK
        pltpu.sync_copy(idx_hbm.at[pl.ds(base, CHUNK)], idx_v)
        pltpu.async_copy(table_hbm.at[idx_v], rows_v, sem).wait()
        pltpu.sync_copy(rows_v, out_hbm.at[pl.ds(base, CHUNK)])
        return carry

    lax.fori_loop(0, n_chunks, body, 0)


@jax.jit
def kernel(X, table):
    S, T = X.shape
    B = S * T
    idx = X.reshape(B).astype(jnp.int32)
    mesh = plsc.VectorSubcoreMesh(core_axis_name="c", subcore_axis_name="s")
    k = functools.partial(
        pl.kernel,
        mesh=mesh,
        out_type=jax.ShapeDtypeStruct((B, DIM), jnp.float32),
        scratch_types=[
            pltpu.VMEM((CHUNK,), jnp.int32),
            pltpu.VMEM((CHUNK, DIM), jnp.float32),
            pltpu.SemaphoreType.DMA,
        ],
        compiler_params=pltpu.CompilerParams(use_tc_tiling_on_sc=False),
    )(_emb_body)
    out = k(table, idx)
    return out.reshape(S, T, DIM)


# speedup vs baseline: 5.1739x; 1.0903x over previous
import functools

import jax
import jax.numpy as jnp
from jax import lax
from jax.experimental import pallas as pl
from jax.experimental.pallas import tpu as pltpu
from jax.experimental.pallas import tpu_sc as plsc

DIM = 64
NC = 2
NS = 16
NW = NC * NS
CHUNK = 800


def _emb_body(table_hbm, idx_hbm, out_hbm,
              idx0, idx1, rows0, rows1, gsem0, gsem1, osem0, osem1):
    wid = lax.axis_index("s") * NC + lax.axis_index("c")
    b_per_w = idx_hbm.shape[0] // NW
    n_chunks = b_per_w // CHUNK
    wbase = wid * b_per_w

    idx_v = (idx0, idx1)
    rows_v = (rows0, rows1)
    gsem = (gsem0, gsem1)
    osem = (osem0, osem1)

    def idx_slice(c):
        return idx_hbm.at[pl.ds(wbase + c * CHUNK, CHUNK)]

    def out_slice(c):
        return out_hbm.at[pl.ds(wbase + c * CHUNK, CHUNK)]

    pltpu.sync_copy(idx_slice(0), idx_v[0])
    pltpu.async_copy(table_hbm.at[idx_v[0]], rows_v[0], gsem[0])

    def outer(j, carry):
        for t in (0, 1):
            c = 2 * j + t
            nt = 1 - t

            @pl.when(c + 1 < n_chunks)
            def _fire_next():
                @pl.when(c >= 1)
                def _drain_prev_write():
                    pltpu.make_async_copy(
                        rows_v[nt], out_slice(c - 1), osem[nt]).wait()
                pltpu.sync_copy(idx_slice(c + 1), idx_v[nt])
                pltpu.async_copy(table_hbm.at[idx_v[nt]], rows_v[nt], gsem[nt])

            pltpu.make_async_copy(
                table_hbm.at[idx_v[t]], rows_v[t], gsem[t]).wait()
            pltpu.async_copy(rows_v[t], out_slice(c), osem[t])
        return carry

    lax.fori_loop(0, n_chunks // 2, outer, 0)

    pltpu.make_async_copy(rows_v[0], out_slice(n_chunks - 2), osem[0]).wait()
    pltpu.make_async_copy(rows_v[1], out_slice(n_chunks - 1), osem[1]).wait()


@jax.jit
def kernel(X, table):
    S, T = X.shape
    B = S * T
    idx = X.reshape(B).astype(jnp.int32)
    mesh = plsc.VectorSubcoreMesh(core_axis_name="c", subcore_axis_name="s")
    k = functools.partial(
        pl.kernel,
        mesh=mesh,
        out_type=jax.ShapeDtypeStruct((B, DIM), jnp.float32),
        scratch_types=[
            pltpu.VMEM((CHUNK,), jnp.int32),
            pltpu.VMEM((CHUNK,), jnp.int32),
            pltpu.VMEM((CHUNK, DIM), jnp.float32),
            pltpu.VMEM((CHUNK, DIM), jnp.float32),
            pltpu.SemaphoreType.DMA,
            pltpu.SemaphoreType.DMA,
            pltpu.SemaphoreType.DMA,
            pltpu.SemaphoreType.DMA,
        ],
        compiler_params=pltpu.CompilerParams(use_tc_tiling_on_sc=False),
    )(_emb_body)
    out = k(table, idx)
    return out.reshape(S, T, DIM)
